# baseline (speedup 1.0000x reference)
import functools

import jax
import jax.numpy as jnp
from jax import lax
from jax.experimental import pallas as pl
from jax.experimental.pallas import tpu as pltpu

N_LAB = 10
T = 20


def _body(lab_sref, x_ref, rates_ref, assign_ref, acc_ref, cnt_ref,
          *, grid, bb):
    i = pl.program_id(0)

    @pl.when(i == 0)
    def _init():
        acc_ref[...] = jnp.zeros_like(acc_ref)
        for l in range(N_LAB):
            cnt_ref[l] = 0

    zero = jnp.zeros((8, 128), jnp.float32)

    def body(b, accs):
        groups = []
        for g in range(T // 4):
            gs = x_ref[b, 4 * g]
            for t in range(4 * g + 1, 4 * g + 4):
                gs = gs + x_ref[b, t]
            groups.append(gs)
        s = groups[0]
        for g in range(1, T // 4):
            s = s + groups[g]

        lab = lab_sref[i * bb + b]
        cnt_ref[lab] = cnt_ref[lab] + 1
        return tuple(
            accs[l] + jnp.where(lab == l, s, zero) for l in range(N_LAB))

    accs0 = tuple(acc_ref[l] for l in range(N_LAB))
    accs = lax.fori_loop(0, bb, body, accs0)
    for l in range(N_LAB):
        acc_ref[l] = accs[l]

    @pl.when(i == grid - 1)
    def _finish():
        means = []
        for l in range(N_LAB):
            c = cnt_ref[l]
            m_l = acc_ref[l] / jnp.maximum(c.astype(jnp.float32), 1.0)
            m_l = jnp.where(c > 0, m_l, 0.0)
            means.append(m_l)
            rates_ref[l] = m_l
        m = means[0]
        am = jnp.zeros(m.shape, dtype=jnp.int32)
        for l in range(1, N_LAB):
            gt = means[l] > m
            am = jnp.where(gt, l, am)
            m = jnp.where(gt, means[l], m)
        assign_ref[...] = am


@jax.jit
def kernel(spikes, labels):
    b, t, n = spikes.shape
    x4 = spikes.reshape(b, t, n // 128, 128)

    grid = 32
    bb = b // grid

    grid_spec = pltpu.PrefetchScalarGridSpec(
        num_scalar_prefetch=1,
        grid=(grid,),
        in_specs=[
            pl.BlockSpec((bb, t, n // 128, 128), lambda i, lab: (i, 0, 0, 0)),
        ],
        out_specs=[
            pl.BlockSpec((N_LAB, n // 128, 128), lambda i, lab: (0, 0, 0)),
            pl.BlockSpec((n // 128, 128), lambda i, lab: (0, 0)),
        ],
        scratch_shapes=[
            pltpu.VMEM((N_LAB, n // 128, 128), jnp.float32),
            pltpu.SMEM((N_LAB,), jnp.int32),
        ],
    )

    rates3, assign2 = pl.pallas_call(
        functools.partial(_body, grid=grid, bb=bb),
        grid_spec=grid_spec,
        out_shape=[
            jax.ShapeDtypeStruct((N_LAB, n // 128, 128), jnp.float32),
            jax.ShapeDtypeStruct((n // 128, 128), jnp.int32),
        ],
    )(labels, x4)

    rates = rates3.reshape(N_LAB, n).T
    assignments = assign2.reshape(n)
    return assignments, rates


# speedup vs baseline: 1.1391x; 1.1391x over previous
import functools

import jax
import jax.numpy as jnp
from jax import lax
from jax.experimental import pallas as pl
from jax.experimental.pallas import tpu as pltpu

N_LAB = 10
T = 20


def _sh(x, k):
    return jnp.concatenate([x[:, k:, :], x[:, :k, :]], axis=1)


def _body(order_sref, starts_sref, counts_sref, x_ref,
          rates_ref, assign_ref, acc_ref, s_ref, *, grid, bb):
    i = pl.program_id(0)

    @pl.when(i == 0)
    def _init():
        acc_ref[...] = jnp.zeros_like(acc_ref)

    n = x_ref.shape[2]

    a = x_ref[:, 0:8, :]
    b4 = x_ref[:, 8:16, :]
    c = x_ref[:, 16:20, :]
    cp = jnp.concatenate([c, jnp.zeros((bb, 4, n), jnp.float32)], axis=1)

    def gtree(x):
        return ((x + _sh(x, 1)) + _sh(x, 2)) + _sh(x, 3)

    ga = gtree(a)
    gb = gtree(b4)
    gc = gtree(cp)
    s = (((ga + _sh(ga, 4)) + gb) + _sh(gb, 4)) + gc
    s_ref[...] = s[:, 0:1, :]

    for l in range(N_LAB):
        start = starts_sref[i * N_LAB + l]
        cnt = counts_sref[i * N_LAB + l]

        def run(k, acc):
            j = order_sref[i * bb + k]
            return acc + s_ref[j, 0, :]

        acc = lax.fori_loop(start, start + cnt, run, acc_ref[l, 0, :])
        acc_ref[l, 0, :] = acc

    @pl.when(i == grid - 1)
    def _finish():
        total = counts_sref[grid * N_LAB]
        means = []
        for l in range(N_LAB):
            c_l = counts_sref[grid * N_LAB + l]
            m_l = acc_ref[l, 0, :] / jnp.maximum(c_l.astype(jnp.float32), 1.0)
            m_l = jnp.where(c_l > 0, m_l, 0.0)
            means.append(m_l)
            rates_ref[l, 0, :] = m_l
        del total
        m = means[0]
        am = jnp.zeros(m.shape, dtype=jnp.int32)
        for l in range(1, N_LAB):
            gt = means[l] > m
            am = jnp.where(gt, l, am)
            m = jnp.where(gt, means[l], m)
        assign_ref[0, :] = am


@jax.jit
def kernel(spikes, labels):
    b, t, n = spikes.shape

    grid = 32
    bb = b // grid

    lab_blk = labels.reshape(grid, bb)
    order_local = jnp.argsort(lab_blk, axis=1, stable=True).astype(jnp.int32)
    counts_blk = jax.vmap(
        lambda v: jnp.bincount(v, length=N_LAB))(lab_blk).astype(jnp.int32)
    starts_blk = jnp.cumsum(counts_blk, axis=1) - counts_blk
    counts_tot = jnp.sum(counts_blk, axis=0, dtype=jnp.int32)
    counts_flat = jnp.concatenate(
        [counts_blk.reshape(-1), counts_tot])

    grid_spec = pltpu.PrefetchScalarGridSpec(
        num_scalar_prefetch=3,
        grid=(grid,),
        in_specs=[
            pl.BlockSpec((bb, t, n), lambda i, *_: (i, 0, 0)),
        ],
        out_specs=[
            pl.BlockSpec((N_LAB, 1, n), lambda i, *_: (0, 0, 0)),
            pl.BlockSpec((1, n), lambda i, *_: (0, 0)),
        ],
        scratch_shapes=[
            pltpu.VMEM((N_LAB, 1, n), jnp.float32),
            pltpu.VMEM((bb, 1, n), jnp.float32),
        ],
    )

    rates3, assign2 = pl.pallas_call(
        functools.partial(_body, grid=grid, bb=bb),
        grid_spec=grid_spec,
        out_shape=[
            jax.ShapeDtypeStruct((N_LAB, 1, n), jnp.float32),
            jax.ShapeDtypeStruct((1, n), jnp.int32),
        ],
    )(order_local.reshape(-1), starts_blk.reshape(-1), counts_flat, spikes)

    rates = rates3.reshape(N_LAB, n).T
    assignments = assign2.reshape(n)
    return assignments, rates
